# Initial kernel scaffold; baseline (speedup 1.0000x reference)
import jax
import jax.numpy as jnp
from jax.experimental import pallas as pl


def kernel(x, edge_index, W1, b1, W2, b2):
    raise NotImplementedError("write your pallas kernel here")


# speedup vs baseline: 3.5166x; 3.5166x over previous
import functools

import jax
import jax.numpy as jnp
from jax import lax
from jax.experimental import pallas as pl
from jax.experimental.pallas import tpu as pltpu
from jax.experimental.pallas import tpu_sc as plsc

N_NODES = 10000
D_IN = 128
N_EDGES = 320000

NUM_CORES = 2
NUM_SUBCORES = 16
NUM_WORKERS = NUM_CORES * NUM_SUBCORES

EDGES_PER_WORKER = N_EDGES // NUM_WORKERS
CHUNK = 400
NUM_CHUNKS = EDGES_PER_WORKER // CHUNK
GATHER_BATCH = 80
NUM_BATCHES = CHUNK // GATHER_BATCH
ROWS_PER_STEP = 4


def _sc_diff_kernel(x, src, dst):
    mesh = plsc.VectorSubcoreMesh(
        core_axis_name="c", subcore_axis_name="s",
        num_cores=NUM_CORES, num_subcores=NUM_SUBCORES)

    @functools.partial(
        pl.kernel,
        out_type=jax.ShapeDtypeStruct((N_EDGES, D_IN), jnp.float32),
        mesh=mesh,
        scratch_types=[
            pltpu.VMEM((CHUNK,), jnp.int32),
            pltpu.VMEM((CHUNK,), jnp.int32),
            pltpu.VMEM((CHUNK, D_IN), jnp.float32),
            pltpu.VMEM((CHUNK, D_IN), jnp.float32),
            pltpu.SemaphoreType.DMA,
        ],
    )
    def k(x_hbm, src_hbm, dst_hbm, diff_hbm, sidx, didx, srows, drows, sem):
        wid = lax.axis_index("s") * NUM_CORES + lax.axis_index("c")
        base = wid * EDGES_PER_WORKER

        def chunk_body(ci, carry):
            cbase = base + ci * CHUNK
            pltpu.sync_copy(src_hbm.at[pl.ds(cbase, CHUNK)], sidx)
            pltpu.sync_copy(dst_hbm.at[pl.ds(cbase, CHUNK)], didx)
            copies = []
            for b in range(NUM_BATCHES):
                sl = pl.ds(b * GATHER_BATCH, GATHER_BATCH)
                copies.append(
                    pltpu.async_copy(x_hbm.at[sidx.at[sl]], srows.at[sl], sem))
                copies.append(
                    pltpu.async_copy(x_hbm.at[didx.at[sl]], drows.at[sl], sem))
            for c in copies:
                c.wait()

            def row_body(i, carry2):
                for rr in range(ROWS_PER_STEP):
                    r = i * ROWS_PER_STEP + rr
                    for k in range(D_IN // 16):
                        s = pl.ds(k * 16, 16)
                        drows[r, s] = jnp.abs(drows[r, s] - srows[r, s])
                return carry2

            lax.fori_loop(0, CHUNK // ROWS_PER_STEP, row_body, 0)
            pltpu.sync_copy(drows, diff_hbm.at[pl.ds(cbase, CHUNK)])
            return carry

        lax.fori_loop(0, NUM_CHUNKS, chunk_body, 0)

    return k(x, src, dst)


BLOCK_E = 3200


def _tc_mlp_body(diff_ref, w1_ref, b1_ref, w2_ref, b2_ref, out_ref):
    d = diff_ref[...]
    h = jnp.dot(d, w1_ref[...], preferred_element_type=jnp.float32)
    h = jnp.maximum(h + b1_ref[...], 0.0)
    e = jnp.dot(h, w2_ref[...], preferred_element_type=jnp.float32)
    out_ref[...] = jax.nn.sigmoid(e + b2_ref[...])


def _tc_mlp(diff, W1, b1, W2, b2):
    grid = (N_EDGES // BLOCK_E,)
    return pl.pallas_call(
        _tc_mlp_body,
        grid=grid,
        in_specs=[
            pl.BlockSpec((BLOCK_E, D_IN), lambda i: (i, 0)),
            pl.BlockSpec((D_IN, 64), lambda i: (0, 0)),
            pl.BlockSpec((1, 64), lambda i: (0, 0)),
            pl.BlockSpec((64, D_IN), lambda i: (0, 0)),
            pl.BlockSpec((1, D_IN), lambda i: (0, 0)),
        ],
        out_specs=pl.BlockSpec((BLOCK_E, D_IN), lambda i: (i, 0)),
        out_shape=jax.ShapeDtypeStruct((N_EDGES, D_IN), jnp.float32),
    )(diff, W1, b1, W2, b2)


def kernel(x, edge_index, W1, b1, W2, b2):
    src = edge_index[0]
    dst = edge_index[1]
    diff = _sc_diff_kernel(x, src, dst)
    return _tc_mlp(diff, W1, b1.reshape(1, 64), W2, b2.reshape(1, 128))


# speedup vs baseline: 4.6651x; 1.3266x over previous
import functools

import jax
import jax.numpy as jnp
from jax import lax
from jax.experimental import pallas as pl
from jax.experimental.pallas import tpu as pltpu
from jax.experimental.pallas import tpu_sc as plsc

N_NODES = 10000
D_IN = 128
N_EDGES = 320000

NUM_CORES = 2
NUM_SUBCORES = 16
NUM_WORKERS = NUM_CORES * NUM_SUBCORES

EDGES_PER_WORKER = N_EDGES // NUM_WORKERS
CHUNK = 200
NUM_CHUNKS = EDGES_PER_WORKER // CHUNK
GATHER_BATCH = 40
NUM_BATCHES = CHUNK // GATHER_BATCH
ROWS_PER_STEP = 4


def _sc_diff_kernel(x, src, dst):
    mesh = plsc.VectorSubcoreMesh(
        core_axis_name="c", subcore_axis_name="s",
        num_cores=NUM_CORES, num_subcores=NUM_SUBCORES)

    @functools.partial(
        pl.kernel,
        out_type=jax.ShapeDtypeStruct((N_EDGES, D_IN), jnp.float32),
        mesh=mesh,
        scratch_types=[
            pltpu.VMEM((EDGES_PER_WORKER,), jnp.int32),
            pltpu.VMEM((EDGES_PER_WORKER,), jnp.int32),
            pltpu.VMEM((CHUNK, D_IN), jnp.float32),
            pltpu.VMEM((CHUNK, D_IN), jnp.float32),
            pltpu.VMEM((CHUNK, D_IN), jnp.float32),
            pltpu.VMEM((CHUNK, D_IN), jnp.float32),
            pltpu.SemaphoreType.DMA,
            pltpu.SemaphoreType.DMA,
            pltpu.SemaphoreType.DMA,
            pltpu.SemaphoreType.DMA,
        ],
    )
    def k(x_hbm, src_hbm, dst_hbm, diff_hbm,
          sidx, didx, srows0, srows1, drows0, drows1,
          sem_g0, sem_g1, sem_w0, sem_w1):
        wid = lax.axis_index("s") * NUM_CORES + lax.axis_index("c")
        base = wid * EDGES_PER_WORKER
        srows = (srows0, srows1)
        drows = (drows0, drows1)
        sem_g = (sem_g0, sem_g1)
        sem_w = (sem_w0, sem_w1)

        def gather_descs(cn, p):
            descs = []
            for b in range(NUM_BATCHES):
                isl = pl.ds(cn * CHUNK + b * GATHER_BATCH, GATHER_BATCH)
                rsl = pl.ds(b * GATHER_BATCH, GATHER_BATCH)
                descs.append((x_hbm.at[sidx.at[isl]], srows[p].at[rsl], sem_g[p]))
                descs.append((x_hbm.at[didx.at[isl]], drows[p].at[rsl], sem_g[p]))
            return descs

        def substep(c, p):
            pp = 1 - p

            @pl.when(c > 0)
            def _():
                pltpu.make_async_copy(
                    drows[pp], diff_hbm.at[pl.ds(base, CHUNK)], sem_w[pp]
                ).wait()

            @pl.when(c + 1 < NUM_CHUNKS)
            def _():
                for s_, d_, sm in gather_descs(c + 1, pp):
                    pltpu.async_copy(s_, d_, sm)

            for s_, d_, sm in gather_descs(c, p):
                pltpu.make_async_copy(s_, d_, sm).wait()

            def row_body(i, carry2):
                for rr in range(ROWS_PER_STEP):
                    r = i * ROWS_PER_STEP + rr
                    for kk in range(D_IN // 16):
                        s = pl.ds(kk * 16, 16)
                        drows[p][r, s] = jnp.abs(drows[p][r, s] - srows[p][r, s])
                return carry2

            lax.fori_loop(0, CHUNK // ROWS_PER_STEP, row_body, 0)

            pltpu.async_copy(
                drows[p], diff_hbm.at[pl.ds(base + c * CHUNK, CHUNK)], sem_w[p])

        pltpu.sync_copy(src_hbm.at[pl.ds(base, EDGES_PER_WORKER)], sidx)
        pltpu.sync_copy(dst_hbm.at[pl.ds(base, EDGES_PER_WORKER)], didx)
        for s_, d_, sm in gather_descs(0, 0):
            pltpu.async_copy(s_, d_, sm)

        def pair_body(i, carry):
            substep(2 * i, 0)
            substep(2 * i + 1, 1)
            return carry

        lax.fori_loop(0, NUM_CHUNKS // 2, pair_body, 0)

        pltpu.make_async_copy(
            drows[1], diff_hbm.at[pl.ds(base, CHUNK)], sem_w[1]).wait()

    return k(x, src, dst)


BLOCK_E = 3200


def _tc_mlp_body(diff_ref, w1_ref, b1_ref, w2_ref, b2_ref, out_ref):
    d = diff_ref[...]
    h = jnp.dot(d, w1_ref[...], preferred_element_type=jnp.float32)
    h = jnp.maximum(h + b1_ref[...], 0.0)
    e = jnp.dot(h, w2_ref[...], preferred_element_type=jnp.float32)
    out_ref[...] = jax.nn.sigmoid(e + b2_ref[...])


def _tc_mlp(diff, W1, b1, W2, b2):
    grid = (N_EDGES // BLOCK_E,)
    return pl.pallas_call(
        _tc_mlp_body,
        grid=grid,
        in_specs=[
            pl.BlockSpec((BLOCK_E, D_IN), lambda i: (i, 0)),
            pl.BlockSpec((D_IN, 64), lambda i: (0, 0)),
            pl.BlockSpec((1, 64), lambda i: (0, 0)),
            pl.BlockSpec((64, D_IN), lambda i: (0, 0)),
            pl.BlockSpec((1, D_IN), lambda i: (0, 0)),
        ],
        out_specs=pl.BlockSpec((BLOCK_E, D_IN), lambda i: (i, 0)),
        out_shape=jax.ShapeDtypeStruct((N_EDGES, D_IN), jnp.float32),
    )(diff, W1, b1, W2, b2)


def kernel(x, edge_index, W1, b1, W2, b2):
    src = edge_index[0]
    dst = edge_index[1]
    diff = _sc_diff_kernel(x, src, dst)
    return _tc_mlp(diff, W1, b1.reshape(1, 64), W2, b2.reshape(1, 128))


# speedup vs baseline: 4.9498x; 1.0610x over previous
import functools

import jax
import jax.numpy as jnp
from jax import lax
from jax.experimental import pallas as pl
from jax.experimental.pallas import tpu as pltpu
from jax.experimental.pallas import tpu_sc as plsc

N_NODES = 10000
D_IN = 128
N_EDGES = 320000

NUM_CORES = 2
NUM_SUBCORES = 16
NUM_WORKERS = NUM_CORES * NUM_SUBCORES

EDGES_PER_WORKER = N_EDGES // NUM_WORKERS
CHUNK = 200
NUM_CHUNKS = EDGES_PER_WORKER // CHUNK
GATHER_BATCH = 40
NUM_BATCHES = CHUNK // GATHER_BATCH
ROWS_PER_STEP = 4


def _sc_diff_kernel(x, src, dst, n_edges):
    mesh = plsc.VectorSubcoreMesh(
        core_axis_name="c", subcore_axis_name="s",
        num_cores=NUM_CORES, num_subcores=NUM_SUBCORES)
    epw = n_edges // NUM_WORKERS
    num_chunks = epw // CHUNK
    assert epw % CHUNK == 0 and num_chunks % 2 == 0

    @functools.partial(
        pl.kernel,
        out_type=jax.ShapeDtypeStruct((n_edges, D_IN), jnp.float32),
        mesh=mesh,
        scratch_types=[
            pltpu.VMEM((epw,), jnp.int32),
            pltpu.VMEM((epw,), jnp.int32),
            pltpu.VMEM((CHUNK, D_IN), jnp.float32),
            pltpu.VMEM((CHUNK, D_IN), jnp.float32),
            pltpu.VMEM((CHUNK, D_IN), jnp.float32),
            pltpu.VMEM((CHUNK, D_IN), jnp.float32),
            pltpu.SemaphoreType.DMA,
            pltpu.SemaphoreType.DMA,
            pltpu.SemaphoreType.DMA,
            pltpu.SemaphoreType.DMA,
        ],
    )
    def k(x_hbm, src_hbm, dst_hbm, diff_hbm,
          sidx, didx, srows0, srows1, drows0, drows1,
          sem_g0, sem_g1, sem_w0, sem_w1):
        wid = lax.axis_index("s") * NUM_CORES + lax.axis_index("c")
        base = wid * epw
        srows = (srows0, srows1)
        drows = (drows0, drows1)
        sem_g = (sem_g0, sem_g1)
        sem_w = (sem_w0, sem_w1)

        def gather_descs(cn, p):
            descs = []
            for b in range(NUM_BATCHES):
                isl = pl.ds(cn * CHUNK + b * GATHER_BATCH, GATHER_BATCH)
                rsl = pl.ds(b * GATHER_BATCH, GATHER_BATCH)
                descs.append((x_hbm.at[sidx.at[isl]], srows[p].at[rsl], sem_g[p]))
                descs.append((x_hbm.at[didx.at[isl]], drows[p].at[rsl], sem_g[p]))
            return descs

        def substep(c, p):
            pp = 1 - p

            @pl.when(c > 0)
            def _():
                pltpu.make_async_copy(
                    drows[pp], diff_hbm.at[pl.ds(base, CHUNK)], sem_w[pp]
                ).wait()

            @pl.when(c + 1 < num_chunks)
            def _():
                for s_, d_, sm in gather_descs(c + 1, pp):
                    pltpu.async_copy(s_, d_, sm)

            for s_, d_, sm in gather_descs(c, p):
                pltpu.make_async_copy(s_, d_, sm).wait()

            def row_body(i, carry2):
                for rr in range(ROWS_PER_STEP):
                    r = i * ROWS_PER_STEP + rr
                    for kk in range(D_IN // 16):
                        s = pl.ds(kk * 16, 16)
                        drows[p][r, s] = jnp.abs(drows[p][r, s] - srows[p][r, s])
                return carry2

            lax.fori_loop(0, CHUNK // ROWS_PER_STEP, row_body, 0)

            pltpu.async_copy(
                drows[p], diff_hbm.at[pl.ds(base + c * CHUNK, CHUNK)], sem_w[p])

        pltpu.sync_copy(src_hbm.at[pl.ds(base, epw)], sidx)
        pltpu.sync_copy(dst_hbm.at[pl.ds(base, epw)], didx)
        for s_, d_, sm in gather_descs(0, 0):
            pltpu.async_copy(s_, d_, sm)

        def pair_body(i, carry):
            substep(2 * i, 0)
            substep(2 * i + 1, 1)
            return carry

        lax.fori_loop(0, num_chunks // 2, pair_body, 0)

        pltpu.make_async_copy(
            drows[1], diff_hbm.at[pl.ds(base, CHUNK)], sem_w[1]).wait()

    return k(x, src, dst)


BLOCK_E = 3200
N_SLICES = 5


def _mlp(diff_ref, w1_ref, b1_ref, w2_ref, b2_ref, out_ref):
    d = diff_ref[...]
    h = jnp.dot(d, w1_ref[...], preferred_element_type=jnp.float32)
    h = jnp.maximum(h + b1_ref[...], 0.0)
    e = jnp.dot(h, w2_ref[...], preferred_element_type=jnp.float32)
    out_ref[...] = jax.nn.sigmoid(e + b2_ref[...])


def _tc_mlp_body(diff_ref, w1_ref, b1_ref, w2_ref, b2_ref, acc_ref, out_ref):
    del acc_ref
    _mlp(diff_ref, w1_ref, b1_ref, w2_ref, b2_ref, out_ref)


_WEIGHT_SPECS = [
    pl.BlockSpec((D_IN, 64), lambda i: (0, 0)),
    pl.BlockSpec((1, 64), lambda i: (0, 0)),
    pl.BlockSpec((64, D_IN), lambda i: (0, 0)),
    pl.BlockSpec((1, D_IN), lambda i: (0, 0)),
]


def _tc_mlp_slice(diff, W1, b1, W2, b2, acc, block_base):
    n_rows = diff.shape[0]
    grid = (n_rows // BLOCK_E,)
    dspec = pl.BlockSpec((BLOCK_E, D_IN), lambda i: (i, 0))
    ospec = pl.BlockSpec((BLOCK_E, D_IN), lambda i: (block_base + i, 0))
    oshape = jax.ShapeDtypeStruct((N_EDGES, D_IN), jnp.float32)
    if acc is None:
        return pl.pallas_call(
            _mlp, grid=grid,
            in_specs=[dspec] + _WEIGHT_SPECS,
            out_specs=ospec, out_shape=oshape,
        )(diff, W1, b1, W2, b2)
    return pl.pallas_call(
        _tc_mlp_body, grid=grid,
        in_specs=[dspec] + _WEIGHT_SPECS + [pl.BlockSpec(memory_space=pl.ANY)],
        out_specs=ospec, out_shape=oshape,
        input_output_aliases={5: 0},
    )(diff, W1, b1, W2, b2, acc)


def kernel(x, edge_index, W1, b1, W2, b2):
    src = edge_index[0]
    dst = edge_index[1]
    b1r = b1.reshape(1, 64)
    b2r = b2.reshape(1, 128)
    es = N_EDGES // N_SLICES
    diffs = [
        _sc_diff_kernel(x, src[k * es:(k + 1) * es], dst[k * es:(k + 1) * es], es)
        for k in range(N_SLICES)
    ]
    acc = None
    for k in range(N_SLICES):
        acc = _tc_mlp_slice(diffs[k], W1, b1r, W2, b2r, acc,
                            k * (es // BLOCK_E))
    return acc
